# baseline (speedup 1.0000x reference)
import functools

import jax
import jax.numpy as jnp
from jax import lax
from jax.experimental import pallas as pl
from jax.experimental.pallas import tpu as pltpu
from jax.experimental.pallas import tpu_sc as plsc

NUM_FIELDS = 26
VOCAB = 100000
HIDDEN = 64
BATCH = 16384

CHUNK = 128
NC = 2
NS = 16
NW = NC * NS
ROWW = 2 * HIDDEN
NGROUP = 2
NFG = NUM_FIELDS // NGROUP

_MESH = plsc.VectorSubcoreMesh(core_axis_name="c", subcore_axis_name="s")


_VBT = 8192


def _tp_body(in_ref, o_ref):
    x = in_ref[...]
    eye2 = jnp.concatenate(
        [jnp.eye(HIDDEN, dtype=jnp.float32)] * 2, axis=1)
    t2 = lax.dot_general(x, eye2, (((0,), (0,)), ((), ())),
                         preferred_element_type=jnp.float32)
    o_ref[0] = t2


def _transpose_tables(tab_t_g):
    out3 = pl.pallas_call(
        _tp_body,
        grid=(NFG, pl.cdiv(VOCAB, _VBT)),
        in_specs=[pl.BlockSpec((HIDDEN, _VBT), lambda i, v: (i, v))],
        out_specs=pl.BlockSpec((1, _VBT, ROWW), lambda i, v: (i, v, 0)),
        out_shape=jax.ShapeDtypeStruct((NFG, VOCAB, ROWW), jnp.float32),
    )(tab_t_g)
    return out3.reshape(NFG * VOCAB, ROWW)


BF_G = BATCH * NFG
N_CHUNKS_G = BF_G // CHUNK
CPW = N_CHUNKS_G // NW


@functools.partial(
    pl.kernel,
    mesh=_MESH,
    out_type=jax.ShapeDtypeStruct((BF_G, HIDDEN), jnp.float32),
    scratch_types=[
        pltpu.VMEM((CPW, CHUNK), jnp.int32),
        pltpu.VMEM((4, CHUNK, ROWW), jnp.float32),
        pltpu.SemaphoreType.DMA,
    ],
    compiler_params=pltpu.CompilerParams(use_tc_tiling_on_sc=False),
)
def _sc_gather(tab_hbm, idx_hbm, out_hbm, idx_v, rows_v, gsem):
    wid = lax.axis_index("s") * NC + lax.axis_index("c")
    cbase = wid * CPW
    pltpu.sync_copy(idx_hbm.at[pl.ds(cbase, CPW)], idx_v)

    def body(jj, carry):
        j0 = jj * 4
        cps = [
            pltpu.async_copy(tab_hbm.at[idx_v.at[j0 + u]], rows_v.at[u], gsem)
            for u in range(4)
        ]
        for cp in cps:
            cp.wait()
        for u in range(4):
            pltpu.sync_copy(
                rows_v.at[u, :, pl.ds(0, HIDDEN)],
                out_hbm.at[pl.ds((cbase + j0 + u) * CHUNK, CHUNK)])
        return carry

    lax.fori_loop(0, CPW // 4, body, 0)


_BM = 8192


def _mm_body(g_ref, w_ref, b_ref, o_ref):
    k = pl.program_id(1)
    acc = jnp.dot(g_ref[0], w_ref[0], preferred_element_type=jnp.float32)

    @pl.when(k == 0)
    def _init():
        o_ref[...] = acc + b_ref[...]

    @pl.when(k != 0)
    def _acc():
        o_ref[...] += acc


def _tc_project(g3, Wd, b2):
    return pl.pallas_call(
        _mm_body,
        grid=(BATCH // 2 // _BM, NFG),
        in_specs=[
            pl.BlockSpec((1, _BM, ROWW), lambda i, k: (k, i, 0)),
            pl.BlockSpec((1, ROWW, ROWW), lambda i, k: (k, 0, 0)),
            pl.BlockSpec((1, ROWW), lambda i, k: (0, 0)),
        ],
        out_specs=pl.BlockSpec((_BM, ROWW), lambda i, k: (i, 0)),
        out_shape=jax.ShapeDtypeStruct((BATCH // 2, ROWW), jnp.float32),
    )(g3, Wd, b2.reshape(1, ROWW))


def kernel(x, tables, W, b):
    tab_t = tables.transpose(0, 2, 1).reshape(NUM_FIELDS * HIDDEN, VOCAB)
    xt = x.T.astype(jnp.int32)
    offs = jnp.arange(NFG, dtype=jnp.int32) * VOCAB
    Wr = W.reshape(NUM_FIELDS, HIDDEN, HIDDEN)
    Wd = jnp.einsum("pq,iab->ipaqb", jnp.eye(2, dtype=jnp.float32),
                    Wr).reshape(NUM_FIELDS, ROWW, ROWW)
    b2 = jnp.concatenate([b, b])
    z2 = jnp.zeros_like(b2)

    outs = []
    for g in range(NGROUP):
        tab_g = _transpose_tables(tab_t[g * NFG * HIDDEN:(g + 1) * NFG * HIDDEN])
        idx_g = (xt[g * NFG:(g + 1) * NFG] + offs[:, None]).reshape(
            N_CHUNKS_G, CHUNK)
        gathered = _sc_gather(tab_g, idx_g)
        g3 = gathered.reshape(NFG, BATCH // 2, ROWW)
        outs.append(_tc_project(g3, Wd[g * NFG:(g + 1) * NFG],
                                b2 if g == 0 else z2))

    out_pairs = outs[0]
    for o in outs[1:]:
        out_pairs = out_pairs + o
    return out_pairs.reshape(BATCH, HIDDEN)


# speedup vs baseline: 1.4393x; 1.4393x over previous
import functools

import jax
import jax.numpy as jnp
from jax import lax
from jax.experimental import pallas as pl
from jax.experimental.pallas import tpu as pltpu
from jax.experimental.pallas import tpu_sc as plsc

NUM_FIELDS = 26
VOCAB = 100000
HIDDEN = 64
BATCH = 16384

CHUNK = 128
NC = 2
NS = 16
NW = NC * NS
ROWW = 2 * HIDDEN
NGROUP = 2
NFG = NUM_FIELDS // NGROUP

_MESH = plsc.VectorSubcoreMesh(core_axis_name="c", subcore_axis_name="s")


_VBT = 8192


def _tp_body(in_ref, o_ref):
    x = in_ref[...]
    eye2 = jnp.concatenate(
        [jnp.eye(HIDDEN, dtype=jnp.float32)] * 2, axis=1)
    t2 = lax.dot_general(x, eye2, (((0,), (0,)), ((), ())),
                         preferred_element_type=jnp.float32)
    o_ref[0] = t2


def _transpose_tables(tab_t, g):
    out3 = pl.pallas_call(
        _tp_body,
        grid=(NFG, pl.cdiv(VOCAB, _VBT)),
        in_specs=[pl.BlockSpec((HIDDEN, _VBT),
                               lambda i, v: (g * NFG + i, v))],
        out_specs=pl.BlockSpec((1, _VBT, ROWW), lambda i, v: (i, v, 0)),
        out_shape=jax.ShapeDtypeStruct((NFG, VOCAB, ROWW), jnp.float32),
    )(tab_t)
    return out3.reshape(NFG * VOCAB, ROWW)


BF_G = BATCH * NFG
N_CHUNKS_G = BF_G // CHUNK
CPW = N_CHUNKS_G // NW


@functools.partial(
    pl.kernel,
    mesh=_MESH,
    out_type=jax.ShapeDtypeStruct((BF_G, HIDDEN), jnp.float32),
    scratch_types=[
        pltpu.VMEM((CPW, CHUNK), jnp.int32),
        pltpu.VMEM((4, CHUNK, ROWW), jnp.float32),
        pltpu.SemaphoreType.DMA,
    ],
    compiler_params=pltpu.CompilerParams(use_tc_tiling_on_sc=False),
)
def _sc_gather(tab_hbm, idx_hbm, out_hbm, idx_v, rows_v, gsem):
    wid = lax.axis_index("s") * NC + lax.axis_index("c")
    cbase = wid * CPW
    pltpu.sync_copy(idx_hbm.at[pl.ds(cbase, CPW)], idx_v)

    def body(jj, carry):
        j0 = jj * 4
        cps = [
            pltpu.async_copy(tab_hbm.at[idx_v.at[j0 + u]], rows_v.at[u], gsem)
            for u in range(4)
        ]
        for cp in cps:
            cp.wait()
        for u in range(4):
            pltpu.sync_copy(
                rows_v.at[u, :, pl.ds(0, HIDDEN)],
                out_hbm.at[pl.ds((cbase + j0 + u) * CHUNK, CHUNK)])
        return carry

    lax.fori_loop(0, CPW // 4, body, 0)


_BM = 8192


def _mm_body(g_ref, w_ref, b_ref, o_ref):
    k = pl.program_id(1)
    acc = jnp.dot(g_ref[0], w_ref[0], preferred_element_type=jnp.float32)

    @pl.when(k == 0)
    def _init():
        o_ref[...] = acc + b_ref[...]

    @pl.when(k != 0)
    def _acc():
        o_ref[...] += acc


def _tc_project(g3, Wd, b2):
    return pl.pallas_call(
        _mm_body,
        grid=(BATCH // 2 // _BM, NFG),
        in_specs=[
            pl.BlockSpec((1, _BM, ROWW), lambda i, k: (k, i, 0)),
            pl.BlockSpec((1, ROWW, ROWW), lambda i, k: (k, 0, 0)),
            pl.BlockSpec((1, ROWW), lambda i, k: (0, 0)),
        ],
        out_specs=pl.BlockSpec((_BM, ROWW), lambda i, k: (i, 0)),
        out_shape=jax.ShapeDtypeStruct((BATCH // 2, ROWW), jnp.float32),
    )(g3, Wd, b2.reshape(1, ROWW))


def kernel(x, tables, W, b):
    tab_t = tables.transpose(0, 2, 1).reshape(NUM_FIELDS * HIDDEN, VOCAB)
    xt = x.T.astype(jnp.int32)
    offs = jnp.arange(NFG, dtype=jnp.int32) * VOCAB
    Wr = W.reshape(NUM_FIELDS, HIDDEN, HIDDEN)
    Wd = jnp.einsum("pq,iab->ipaqb", jnp.eye(2, dtype=jnp.float32),
                    Wr).reshape(NUM_FIELDS, ROWW, ROWW)
    b2 = jnp.concatenate([b, b])
    z2 = jnp.zeros_like(b2)

    outs = []
    for g in range(NGROUP):
        tab_g = _transpose_tables(tab_t, g)
        idx_g = (xt[g * NFG:(g + 1) * NFG] + offs[:, None]).reshape(
            N_CHUNKS_G, CHUNK)
        gathered = _sc_gather(tab_g, idx_g)
        g3 = gathered.reshape(NFG, BATCH // 2, ROWW)
        outs.append(_tc_project(g3, Wd[g * NFG:(g + 1) * NFG],
                                b2 if g == 0 else z2))

    out_pairs = outs[0]
    for o in outs[1:]:
        out_pairs = out_pairs + o
    return out_pairs.reshape(BATCH, HIDDEN)


# speedup vs baseline: 1.5304x; 1.0633x over previous
import functools

import jax
import jax.numpy as jnp
from jax import lax
from jax.experimental import pallas as pl
from jax.experimental.pallas import tpu as pltpu
from jax.experimental.pallas import tpu_sc as plsc

NUM_FIELDS = 26
VOCAB = 100000
HIDDEN = 64
BATCH = 16384

CHUNK = 128
NC = 2
NS = 16
NW = NC * NS
ROWW = 2 * HIDDEN
NGROUP = 2
NFG = NUM_FIELDS // NGROUP

_MESH = plsc.VectorSubcoreMesh(core_axis_name="c", subcore_axis_name="s")


_VBT = 16384


def _tp_body(in_ref, o_ref):
    x = in_ref[...]
    eye2 = jnp.concatenate(
        [jnp.eye(HIDDEN, dtype=jnp.float32)] * 2, axis=1)
    t2 = lax.dot_general(x, eye2, (((0,), (0,)), ((), ())),
                         preferred_element_type=jnp.float32)
    o_ref[0] = t2


def _transpose_tables(tab_t, g):
    out3 = pl.pallas_call(
        _tp_body,
        grid=(NFG, pl.cdiv(VOCAB, _VBT)),
        in_specs=[pl.BlockSpec((HIDDEN, _VBT),
                               lambda i, v: (g * NFG + i, v))],
        out_specs=pl.BlockSpec((1, _VBT, ROWW), lambda i, v: (i, v, 0)),
        out_shape=jax.ShapeDtypeStruct((NFG, VOCAB, ROWW), jnp.float32),
    )(tab_t)
    return out3.reshape(NFG * VOCAB, ROWW)


BF_G = BATCH * NFG
N_CHUNKS_G = BF_G // CHUNK
CPW = N_CHUNKS_G // NW


@functools.partial(
    pl.kernel,
    mesh=_MESH,
    out_type=jax.ShapeDtypeStruct((BF_G, HIDDEN), jnp.float32),
    scratch_types=[
        pltpu.VMEM((CPW, CHUNK), jnp.int32),
        pltpu.VMEM((4, CHUNK, ROWW), jnp.float32),
        pltpu.SemaphoreType.DMA,
    ],
    compiler_params=pltpu.CompilerParams(use_tc_tiling_on_sc=False),
)
def _sc_gather(tab_hbm, idx_hbm, out_hbm, idx_v, rows_v, gsem):
    wid = lax.axis_index("s") * NC + lax.axis_index("c")
    cbase = wid * CPW
    pltpu.sync_copy(idx_hbm.at[pl.ds(cbase, CPW)], idx_v)

    def body(jj, carry):
        j0 = jj * 4
        cps = [
            pltpu.async_copy(tab_hbm.at[idx_v.at[j0 + u]], rows_v.at[u], gsem)
            for u in range(4)
        ]
        for cp in cps:
            cp.wait()
        for u in range(4):
            pltpu.sync_copy(
                rows_v.at[u, :, pl.ds(0, HIDDEN)],
                out_hbm.at[pl.ds((cbase + j0 + u) * CHUNK, CHUNK)])
        return carry

    lax.fori_loop(0, CPW // 4, body, 0)


_BM = 8192


def _mm_body(g_ref, w_ref, b_ref, o_ref):
    k = pl.program_id(1)
    acc = jnp.dot(g_ref[0], w_ref[0], preferred_element_type=jnp.float32)

    @pl.when(k == 0)
    def _init():
        o_ref[...] = acc + b_ref[...]

    @pl.when(k != 0)
    def _acc():
        o_ref[...] += acc


def _tc_project(g3, Wd, b2):
    return pl.pallas_call(
        _mm_body,
        grid=(BATCH // 2 // _BM, NFG),
        in_specs=[
            pl.BlockSpec((1, _BM, ROWW), lambda i, k: (k, i, 0)),
            pl.BlockSpec((1, ROWW, ROWW), lambda i, k: (k, 0, 0)),
            pl.BlockSpec((1, ROWW), lambda i, k: (0, 0)),
        ],
        out_specs=pl.BlockSpec((_BM, ROWW), lambda i, k: (i, 0)),
        out_shape=jax.ShapeDtypeStruct((BATCH // 2, ROWW), jnp.float32),
    )(g3, Wd, b2.reshape(1, ROWW))


def kernel(x, tables, W, b):
    tab_t = tables.transpose(0, 2, 1).reshape(NUM_FIELDS * HIDDEN, VOCAB)
    xt = x.T.astype(jnp.int32)
    offs = jnp.arange(NFG, dtype=jnp.int32) * VOCAB
    Wr = W.reshape(NUM_FIELDS, HIDDEN, HIDDEN)
    Wd = jnp.einsum("pq,iab->ipaqb", jnp.eye(2, dtype=jnp.float32),
                    Wr).reshape(NUM_FIELDS, ROWW, ROWW)
    b2 = jnp.concatenate([b, b])
    z2 = jnp.zeros_like(b2)

    outs = []
    for g in range(NGROUP):
        tab_g = _transpose_tables(tab_t, g)
        idx_g = (xt[g * NFG:(g + 1) * NFG] + offs[:, None]).reshape(
            N_CHUNKS_G, CHUNK)
        gathered = _sc_gather(tab_g, idx_g)
        g3 = gathered.reshape(NFG, BATCH // 2, ROWW)
        outs.append(_tc_project(g3, Wd[g * NFG:(g + 1) * NFG],
                                b2 if g == 0 else z2))

    out_pairs = outs[0]
    for o in outs[1:]:
        out_pairs = out_pairs + o
    return out_pairs.reshape(BATCH, HIDDEN)


# speedup vs baseline: 1.6401x; 1.0717x over previous
import functools

import jax
import jax.numpy as jnp
from jax import lax
from jax.experimental import pallas as pl
from jax.experimental.pallas import tpu as pltpu
from jax.experimental.pallas import tpu_sc as plsc

NUM_FIELDS = 26
VOCAB = 100000
HIDDEN = 64
BATCH = 16384

CHUNK = 128
NC = 2
NS = 16
NW = NC * NS
ROWW = 2 * HIDDEN
NGROUP = 2
NFG = NUM_FIELDS // NGROUP

_MESH = plsc.VectorSubcoreMesh(core_axis_name="c", subcore_axis_name="s")


_VBT = 25088


def _tp_body(in_ref, o_ref):
    x = in_ref[...]
    eye2 = jnp.concatenate(
        [jnp.eye(HIDDEN, dtype=jnp.float32)] * 2, axis=1)
    t2 = lax.dot_general(x, eye2, (((0,), (0,)), ((), ())),
                         preferred_element_type=jnp.float32)
    o_ref[0] = t2


def _transpose_tables(tab_t, g):
    out3 = pl.pallas_call(
        _tp_body,
        grid=(NFG, pl.cdiv(VOCAB, _VBT)),
        in_specs=[pl.BlockSpec((HIDDEN, _VBT),
                               lambda i, v: (g * NFG + i, v))],
        out_specs=pl.BlockSpec((1, _VBT, ROWW), lambda i, v: (i, v, 0)),
        out_shape=jax.ShapeDtypeStruct((NFG, VOCAB, ROWW), jnp.float32),
    )(tab_t)
    return out3.reshape(NFG * VOCAB, ROWW)


BF_G = BATCH * NFG
N_CHUNKS_G = BF_G // CHUNK
CPW = N_CHUNKS_G // NW


@functools.partial(
    pl.kernel,
    mesh=_MESH,
    out_type=jax.ShapeDtypeStruct((BF_G, HIDDEN), jnp.float32),
    scratch_types=[
        pltpu.VMEM((CPW, CHUNK), jnp.int32),
        pltpu.VMEM((4, CHUNK, ROWW), jnp.float32),
        pltpu.SemaphoreType.DMA,
    ],
    compiler_params=pltpu.CompilerParams(use_tc_tiling_on_sc=False),
)
def _sc_gather(tab_hbm, idx_hbm, out_hbm, idx_v, rows_v, gsem):
    wid = lax.axis_index("s") * NC + lax.axis_index("c")
    cbase = wid * CPW
    pltpu.sync_copy(idx_hbm.at[pl.ds(cbase, CPW)], idx_v)

    def body(jj, carry):
        j0 = jj * 4
        cps = [
            pltpu.async_copy(tab_hbm.at[idx_v.at[j0 + u]], rows_v.at[u], gsem)
            for u in range(4)
        ]
        for cp in cps:
            cp.wait()
        for u in range(4):
            pltpu.sync_copy(
                rows_v.at[u, :, pl.ds(0, HIDDEN)],
                out_hbm.at[pl.ds((cbase + j0 + u) * CHUNK, CHUNK)])
        return carry

    lax.fori_loop(0, CPW // 4, body, 0)


_BM = 8192


def _mm_body(g_ref, w_ref, b_ref, o_ref):
    k = pl.program_id(1)
    acc = jnp.dot(g_ref[0], w_ref[0], preferred_element_type=jnp.float32)

    @pl.when(k == 0)
    def _init():
        o_ref[...] = acc + b_ref[...]

    @pl.when(k != 0)
    def _acc():
        o_ref[...] += acc


def _tc_project(g3, Wd, b2):
    return pl.pallas_call(
        _mm_body,
        grid=(BATCH // 2 // _BM, NFG),
        in_specs=[
            pl.BlockSpec((1, _BM, ROWW), lambda i, k: (k, i, 0)),
            pl.BlockSpec((1, ROWW, ROWW), lambda i, k: (k, 0, 0)),
            pl.BlockSpec((1, ROWW), lambda i, k: (0, 0)),
        ],
        out_specs=pl.BlockSpec((_BM, ROWW), lambda i, k: (i, 0)),
        out_shape=jax.ShapeDtypeStruct((BATCH // 2, ROWW), jnp.float32),
    )(g3, Wd, b2.reshape(1, ROWW))


def kernel(x, tables, W, b):
    tab_t = tables.transpose(0, 2, 1).reshape(NUM_FIELDS * HIDDEN, VOCAB)
    xt = x.T.astype(jnp.int32)
    offs = jnp.arange(NFG, dtype=jnp.int32) * VOCAB
    Wr = W.reshape(NUM_FIELDS, HIDDEN, HIDDEN)
    Wd = jnp.einsum("pq,iab->ipaqb", jnp.eye(2, dtype=jnp.float32),
                    Wr).reshape(NUM_FIELDS, ROWW, ROWW)
    b2 = jnp.concatenate([b, b])
    z2 = jnp.zeros_like(b2)

    outs = []
    for g in range(NGROUP):
        tab_g = _transpose_tables(tab_t, g)
        idx_g = (xt[g * NFG:(g + 1) * NFG] + offs[:, None]).reshape(
            N_CHUNKS_G, CHUNK)
        gathered = _sc_gather(tab_g, idx_g)
        g3 = gathered.reshape(NFG, BATCH // 2, ROWW)
        outs.append(_tc_project(g3, Wd[g * NFG:(g + 1) * NFG],
                                b2 if g == 0 else z2))

    out_pairs = outs[0]
    for o in outs[1:]:
        out_pairs = out_pairs + o
    return out_pairs.reshape(BATCH, HIDDEN)
